# Initial kernel scaffold; baseline (speedup 1.0000x reference)
import jax
import jax.numpy as jnp
from jax.experimental import pallas as pl


def kernel(x_user, edge_index_follow, edge_index_friend, des, tweets, Kw, Kb, Qw, Qb, Vw, Vb, Aw, Ab, skip, Arel, Mrel, Prel, proj_w, proj_b, prop_w1, prop_b1, prop_w2, prop_b2, clf_w1, clf_b1, clf_w2, clf_b2):
    raise NotImplementedError("write your pallas kernel here")


# speedup vs baseline: 2.6423x; 2.6423x over previous
import functools
import math

import jax
import jax.numpy as jnp
from jax import lax
from jax.experimental import pallas as pl
from jax.experimental.pallas import tpu as pltpu

N = 10000
C = 128
E = 160000
B = 4096
HID = 256
TXT = 768
D = 128

NP = 10240
EP = 327680
ACC_W = 144


def _leaky(x):
    return jnp.where(x >= 0, x, 0.01 * x)


def _pre_body(h_ref, kw, kb, qw, qb, vw, vb, a0, a1, m0, m1,
              ka0_o, ka1_o, q_o, va0_o, va1_o):
    h = h_ref[...]
    k = jnp.dot(h, kw[...], preferred_element_type=jnp.float32) + kb[...]
    q = jnp.dot(h, qw[...], preferred_element_type=jnp.float32) + qb[...]
    v = jnp.dot(h, vw[...], preferred_element_type=jnp.float32) + vb[...]
    ka0_o[...] = jnp.dot(k, a0[...], preferred_element_type=jnp.float32)
    ka1_o[...] = jnp.dot(k, a1[...], preferred_element_type=jnp.float32)
    q_o[...] = q
    va0_o[...] = jnp.dot(v, m0[...], preferred_element_type=jnp.float32)
    va1_o[...] = jnp.dot(v, m1[...], preferred_element_type=jnp.float32)


def _tc_pre(h, kw, kb, qw, qb, vw, vb, a0s, a1s, m0, m1):
    R = 2000
    grid = (N // R,)
    row = pl.BlockSpec((R, C), lambda i: (i, 0))
    full = pl.BlockSpec((C, C), lambda i: (0, 0))
    bias = pl.BlockSpec((1, C), lambda i: (0, 0))
    out = jax.ShapeDtypeStruct((N, C), jnp.float32)
    return pl.pallas_call(
        _pre_body,
        grid=grid,
        in_specs=[row, full, bias, full, bias, full, bias, full, full, full, full],
        out_specs=[row, row, row, row, row],
        out_shape=[out, out, out, out, out],
    )(h, kw, kb.reshape(1, C), qw, qb.reshape(1, C), vw, vb.reshape(1, C),
      a0s, a1s, m0, m1)


def _post_body(accA, accB, h_ref, aw, ab, s_ref, o_ref):
    a = accA[...] + accB[...]
    den = a[:, 128:129]
    num = a[:, :128]
    out = jnp.where(den > 0, num / jnp.where(den > 0, den, 1.0), 0.0)
    g = 0.5 * out * (1.0 + lax.erf(out * (1.0 / math.sqrt(2.0))))
    upd = jnp.dot(g, aw[...], preferred_element_type=jnp.float32) + ab[...]
    s = s_ref[0, 0]
    o_ref[...] = s * upd + (1.0 - s) * h_ref[...]


def _tc_post(accA, accB, h, aw, ab, s):
    R = 2000
    grid = (N // R,)
    rowa = pl.BlockSpec((R, ACC_W), lambda i: (i, 0))
    row = pl.BlockSpec((R, C), lambda i: (i, 0))
    full = pl.BlockSpec((C, C), lambda i: (0, 0))
    bias = pl.BlockSpec((1, C), lambda i: (0, 0))
    sspec = pl.BlockSpec(memory_space=pltpu.SMEM)
    return pl.pallas_call(
        _post_body,
        grid=grid,
        in_specs=[rowa, rowa, row, full, bias, sspec],
        out_specs=row,
        out_shape=jax.ShapeDtypeStruct((N, C), jnp.float32),
    )(accA, accB, h, aw, ab.reshape(1, C), s.reshape(1, 1))


def _head_body(h_ref, x_ref, des_ref, tw_ref, pw, pb, w1, b1, w2, b2,
               cw1, cb1, cw2, cb2, o_ref):
    ge = _leaky(jnp.dot(h_ref[...], pw[...], preferred_element_type=jnp.float32)
                + pb[...])
    pe0 = _leaky(jnp.dot(x_ref[...], w1[...], preferred_element_type=jnp.float32)
                 + b1[...])
    pe = jnp.dot(pe0, w2[...], preferred_element_type=jnp.float32) + b2[...]
    w = cw1[...]
    t = (jnp.dot(ge, w[:HID], preferred_element_type=jnp.float32)
         + jnp.dot(pe, w[HID:2 * HID], preferred_element_type=jnp.float32)
         + jnp.dot(des_ref[...], w[2 * HID:2 * HID + TXT],
                   preferred_element_type=jnp.float32)
         + jnp.dot(tw_ref[...], w[2 * HID + TXT:],
                   preferred_element_type=jnp.float32)
         + cb1[...])
    l1 = _leaky(t)
    logits = _leaky(jnp.dot(l1, cw2[...], preferred_element_type=jnp.float32)
                    + cb2[...])
    m = jnp.max(logits, axis=-1, keepdims=True)
    e = jnp.exp(logits - m)
    o_ref[...] = e / jnp.sum(e, axis=-1, keepdims=True)


def _tc_head(h, x, des, tweets, pw, pb, w1, b1, w2, b2, cw1, cb1, cw2, cb2):
    R = 1024
    grid = (B // R,)
    row = pl.BlockSpec((R, C), lambda i: (i, 0))
    txt = pl.BlockSpec((R, TXT), lambda i: (i, 0))
    o = pl.BlockSpec((R, 2), lambda i: (i, 0))

    def fixed(shape):
        return pl.BlockSpec(shape, lambda i: tuple(0 for _ in shape))

    return pl.pallas_call(
        _head_body,
        grid=grid,
        in_specs=[row, row, txt, txt,
                  fixed((C, HID)), fixed((1, HID)),
                  fixed((C, HID)), fixed((1, HID)),
                  fixed((HID, HID)), fixed((1, HID)),
                  fixed((2 * HID + 2 * TXT, HID)), fixed((1, HID)),
                  fixed((HID, 2)), fixed((1, 2))],
        out_specs=o,
        out_shape=jax.ShapeDtypeStruct((B, 2), jnp.float32),
    )(h[:B], x[:B], des, tweets, pw, pb.reshape(1, HID),
      w1, b1.reshape(1, HID), w2, b2.reshape(1, HID),
      cw1, cb1.reshape(1, HID), cw2, cb2.reshape(1, 2))


def _edge_phase_jax(ka0, ka1, q, va0, va1, src, dst):
    KA = jnp.concatenate([ka0, ka1], axis=0)
    VA = jnp.concatenate([va0, va1], axis=0)
    al = (KA[src] * q[dst]).sum(-1)
    M = jnp.max(al)
    p = jnp.exp(al - M)
    den = jax.ops.segment_sum(p, dst, num_segments=N)
    num = jax.ops.segment_sum(p[:, None] * VA[src], dst, num_segments=N)
    accA = jnp.zeros((NP, ACC_W), jnp.float32)
    accA = accA.at[:N, :C].set(num)
    accA = accA.at[:N, C].set(den)
    accB = jnp.zeros((NP, ACC_W), jnp.float32)
    return accA, accB


def kernel(x_user, edge_index_follow, edge_index_friend, des, tweets,
           Kw, Kb, Qw, Qb, Vw, Vb, Aw, Ab, skip, Arel, Mrel, Prel,
           proj_w, proj_b, prop_w1, prop_b1, prop_w2, prop_b2,
           clf_w1, clf_b1, clf_w2, clf_b2):
    sf, df = edge_index_follow[0], edge_index_follow[1]
    sr, dr = edge_index_friend[0], edge_index_friend[1]
    src = jnp.concatenate([sf, sr + N])
    dst = jnp.concatenate([df, dr])
    s_skip = jax.nn.sigmoid(skip)

    h = x_user
    for i in range(2):
        sc = Prel[i] / math.sqrt(D)
        a0s = Arel[i, 0] * sc[0]
        a1s = Arel[i, 1] * sc[1]
        ka0, ka1, q, va0, va1 = _tc_pre(
            h, Kw[i], Kb[i], Qw[i], Qb[i], Vw[i], Vb[i],
            a0s, a1s, Mrel[i, 0], Mrel[i, 1])
        accA, accB = _edge_phase_jax(ka0, ka1, q, va0, va1, src, dst)
        h = _tc_post(accA, accB, h, Aw[i], Ab[i], s_skip[i])

    return _tc_head(h, x_user, des, tweets, proj_w, proj_b,
                    prop_w1, prop_b1, prop_w2, prop_b2,
                    clf_w1, clf_b1, clf_w2, clf_b2)
